# baseline (speedup 1.0000x reference)
import functools

import jax
import jax.numpy as jnp
from jax import lax
from jax.experimental import pallas as pl
from jax.experimental.pallas import tpu as pltpu
from jax.experimental.pallas import tpu_sc as plsc

N_NODES = 10000
N_EDGES = 320000
D = 128

NC = 2
NS = 16
L = 16

R = 10112
CHUNK = 64
NCH = 160
CPB = 16
NBLK = NCH // CPB
NBUF = 5
E_PAD = NC * NS * NCH * CHUNK
ROWS_PER_TILE = R // NS


def _seg_sum_body(x_hbm, srcs_hbm, dsts_hbm, out_hbm,
                  src_a, dst_a, src_b, dst_b,
                  r0, r1, r2, r3, r4, acc,
                  g0, g1, g2, g3, g4, s0, s1, s2, s3, s4, ia, ib):
    c = lax.axis_index("c")
    tid = lax.axis_index("s")
    wid = c * NS + tid
    rows = (r0, r1, r2, r3, r4)
    gsem = (g0, g1, g2, g3, g4)
    ssem = (s0, s1, s2, s3, s4)
    slots = ((src_a, dst_a, ia), (src_b, dst_b, ib))

    def stage(B):
        sv, dv, sem = slots[B % 2]
        base = wid * NCH + B * CPB
        pltpu.async_copy(srcs_hbm.at[pl.ds(base, CPB)], sv, sem)
        pltpu.async_copy(dsts_hbm.at[pl.ds(base, CPB)], dv, sem)

    def wait_stage(B):
        sv, dv, sem = slots[B % 2]
        pltpu.make_async_copy(srcs_hbm.at[pl.ds(0, CPB)], sv, sem).wait()
        pltpu.make_async_copy(dsts_hbm.at[pl.ds(0, CPB)], dv, sem).wait()

    def iref(k):
        sv, dv, _ = slots[(k // CPB) % 2]
        return sv.at[k % CPB], dv.at[k % CPB]

    def start_gather(k):
        si, _ = iref(k)
        pltpu.async_copy(x_hbm.at[si], rows[k % NBUF], gsem[k % NBUF])

    def wait_gather(k):
        si, _ = iref(k)
        pltpu.make_async_copy(x_hbm.at[si], rows[k % NBUF],
                              gsem[k % NBUF]).wait()

    def start_scatter(k):
        _, di = iref(k)
        pltpu.async_copy(rows[k % NBUF], acc.at[di], ssem[k % NBUF], add=True)

    def wait_scatter(k):
        _, di = iref(k)
        pltpu.make_async_copy(rows[k % NBUF], acc.at[di],
                              ssem[k % NBUF]).wait()

    stage(0)
    stage(1)

    zero = jnp.zeros((L,), jnp.float32)

    def zrow(i, _):
        for j in range(D // L):
            r0[i, pl.ds(j * L, L)] = zero
        return 0

    lax.fori_loop(0, CHUNK, zrow, 0)
    wait_stage(0)
    start_gather(1)
    start_gather(2)
    row0 = tid * ROWS_PER_TILE
    rem = ROWS_PER_TILE % CHUNK
    nseg0 = ROWS_PER_TILE // CHUNK + (1 if rem else 0)
    for k in range(nseg0):
        nr = CHUNK if k < ROWS_PER_TILE // CHUNK else rem
        r = row0 + k * CHUNK
        xbuf = rows[3 + k % 2].at[pl.ds(0, nr)]

        @pl.when(k % 2 == c)
        def _(xbuf=xbuf, r=r, nr=nr):
            pltpu.sync_copy(x_hbm.at[pl.ds(r, nr)], xbuf)
            pltpu.sync_copy(xbuf, acc.at[pl.ds(r, nr)])

        @pl.when(k % 2 != c)
        def _(r=r, nr=nr):
            pltpu.sync_copy(r0.at[pl.ds(0, nr)], acc.at[pl.ds(r, nr)])

    start_gather(0)
    plsc.subcore_barrier()

    for k in range(NCH):
        blk, off = divmod(k, CPB)
        if off == CPB - 3 and blk + 1 < NBLK:
            wait_stage(blk + 1)
        wait_gather(k)
        start_scatter(k)
        if off == 4 and 1 <= blk and blk + 1 < NBLK:
            stage(blk + 1)
        if k + 3 < NCH:
            if k - 2 >= 0:
                wait_scatter(k - 2)
            start_gather(k + 3)
    for k in range(NCH - 5, NCH):
        wait_scatter(k)
    plsc.subcore_barrier()

    nseg = ROWS_PER_TILE // CHUNK + (1 if rem else 0)
    pushes = []
    for k in range(nseg):
        nr = CHUNK if k < ROWS_PER_TILE // CHUNK else rem
        r = row0 + k * CHUNK
        buf = rows[k % 2].at[pl.ds(0, nr)]
        sem = gsem[k % 2]
        if k >= 2:
            pushes[k - 2].wait()
        pltpu.sync_copy(acc.at[pl.ds(r, nr)], buf)
        pushes.append(
            pltpu.async_copy(buf, out_hbm.at[c, pl.ds(r, nr)], sem))
    for p in pushes[-2:]:
        p.wait()


_seg_sum = pl.kernel(
    _seg_sum_body,
    out_type=jax.ShapeDtypeStruct((NC, R, D), jnp.float32),
    mesh=plsc.VectorSubcoreMesh(core_axis_name="c", subcore_axis_name="s",
                                num_cores=NC, num_subcores=NS),
    scratch_types=[
        pltpu.VMEM((CPB, CHUNK), jnp.int32),
        pltpu.VMEM((CPB, CHUNK), jnp.int32),
        pltpu.VMEM((CPB, CHUNK), jnp.int32),
        pltpu.VMEM((CPB, CHUNK), jnp.int32),
        pltpu.VMEM((CHUNK, D), jnp.float32),
        pltpu.VMEM((CHUNK, D), jnp.float32),
        pltpu.VMEM((CHUNK, D), jnp.float32),
        pltpu.VMEM((CHUNK, D), jnp.float32),
        pltpu.VMEM((CHUNK, D), jnp.float32),
        pltpu.VMEM_SHARED((R, D), jnp.float32),
        pltpu.SemaphoreType.DMA,
        pltpu.SemaphoreType.DMA,
        pltpu.SemaphoreType.DMA,
        pltpu.SemaphoreType.DMA,
        pltpu.SemaphoreType.DMA,
        pltpu.SemaphoreType.DMA,
        pltpu.SemaphoreType.DMA,
        pltpu.SemaphoreType.DMA,
        pltpu.SemaphoreType.DMA,
        pltpu.SemaphoreType.DMA,
        pltpu.SemaphoreType.DMA,
        pltpu.SemaphoreType.DMA,
    ],
)


def _mlp_body(p_ref, w_ref, b_ref, o_ref, *, relu):
    acc = p_ref[0] + p_ref[1]
    h = jnp.dot(acc, w_ref[...], preferred_element_type=jnp.float32)
    h = h + b_ref[...]
    if relu:
        h = jnp.maximum(h, 0.0)
    o_ref[...] = h


def _mlp(partials, w, b2d, relu, rows_out, blk):
    grid = (rows_out // blk,)
    return pl.pallas_call(
        functools.partial(_mlp_body, relu=relu),
        grid=grid,
        in_specs=[
            pl.BlockSpec((NC, blk, D), lambda i: (0, i, 0)),
            pl.BlockSpec((D, D), lambda i: (0, 0)),
            pl.BlockSpec((1, D), lambda i: (0, 0)),
        ],
        out_specs=pl.BlockSpec((blk, D), lambda i: (i, 0)),
        out_shape=jax.ShapeDtypeStruct((rows_out, D), jnp.float32),
    )(partials, w, b2d)


def kernel(x, edge_index, W1, b1, W2, b2):
    src = edge_index[0].astype(jnp.int32)
    dst = edge_index[1].astype(jnp.int32)
    pad = E_PAD - N_EDGES
    pad_ar = jnp.arange(pad, dtype=jnp.int32)
    srcs = jnp.concatenate([src, pad_ar % N_NODES]).reshape(-1, CHUNK)
    dsts = jnp.concatenate([dst, N_NODES + pad_ar % (R - N_NODES)]).reshape(-1, CHUNK)
    b1r = b1.reshape(1, D)
    b2r = b2.reshape(1, D)
    x_pad = jnp.concatenate([x, jnp.zeros((R - N_NODES, D), jnp.float32)])

    p1 = _seg_sum(x_pad, srcs, dsts)
    h = _mlp(p1, W1, b1r, relu=True, rows_out=R, blk=R // 8)
    p2 = _seg_sum(h, srcs, dsts)
    return _mlp(p2, W2, b2r, relu=False, rows_out=N_NODES, blk=2000)


# speedup vs baseline: 1.0556x; 1.0556x over previous
import functools

import jax
import jax.numpy as jnp
from jax import lax
from jax.experimental import pallas as pl
from jax.experimental.pallas import tpu as pltpu
from jax.experimental.pallas import tpu_sc as plsc

N_NODES = 10000
N_EDGES = 320000
D = 128

NC = 2
NS = 16
L = 16

R = 10112
CHUNK = 64
NCH = 160
CPB = 16
NBLK = NCH // CPB
NBUF = 5
E_PAD = NC * NS * NCH * CHUNK
ROWS_PER_TILE = R // NS


def _seg_sum_body(x_hbm, srcs_hbm, dsts_hbm, out_hbm,
                  src_a, dst_a, src_b, dst_b,
                  r0, r1, r2, r3, r4, acc,
                  g0, g1, g2, g3, g4, s0, s1, s2, s3, s4, ia, ib):
    c = lax.axis_index("c")
    tid = lax.axis_index("s")
    wid = c * NS + tid
    rows = (r0, r1, r2, r3, r4)
    gsem = (g0, g1, g2, g3, g4)
    ssem = (s0, s1, s2, s3, s4)
    slots = ((src_a, dst_a, ia), (src_b, dst_b, ib))

    def stage(B):
        sv, dv, sem = slots[B % 2]
        base = wid * NCH + B * CPB
        pltpu.async_copy(srcs_hbm.at[pl.ds(base, CPB)], sv, sem)
        pltpu.async_copy(dsts_hbm.at[pl.ds(base, CPB)], dv, sem)

    def wait_stage(B):
        sv, dv, sem = slots[B % 2]
        pltpu.make_async_copy(srcs_hbm.at[pl.ds(0, CPB)], sv, sem).wait()
        pltpu.make_async_copy(dsts_hbm.at[pl.ds(0, CPB)], dv, sem).wait()

    def iref(k):
        sv, dv, _ = slots[(k // CPB) % 2]
        return sv.at[k % CPB], dv.at[k % CPB]

    def start_gather(k):
        si, _ = iref(k)
        pltpu.async_copy(x_hbm.at[si], rows[k % NBUF], gsem[k % NBUF])

    def wait_gather(k):
        si, _ = iref(k)
        pltpu.make_async_copy(x_hbm.at[si], rows[k % NBUF],
                              gsem[k % NBUF]).wait()

    def start_scatter(k):
        _, di = iref(k)
        pltpu.async_copy(rows[k % NBUF], acc.at[di], ssem[k % NBUF], add=True)

    def wait_scatter(k):
        _, di = iref(k)
        pltpu.make_async_copy(rows[k % NBUF], acc.at[di],
                              ssem[k % NBUF]).wait()

    stage(0)
    stage(1)

    zero = jnp.zeros((L,), jnp.float32)

    def zrow(i, _):
        for j in range(D // L):
            r0[i, pl.ds(j * L, L)] = zero
        return 0

    lax.fori_loop(0, CHUNK, zrow, 0)
    wait_stage(0)
    start_gather(1)
    start_gather(2)
    row0 = tid * ROWS_PER_TILE
    rem = ROWS_PER_TILE % CHUNK
    for k in range(ROWS_PER_TILE // CHUNK):
        pltpu.sync_copy(r0, acc.at[pl.ds(row0 + k * CHUNK, CHUNK)])
    if rem:
        pltpu.sync_copy(r0.at[pl.ds(0, rem)],
                        acc.at[pl.ds(row0 + ROWS_PER_TILE - rem, rem)])
    start_gather(0)
    plsc.subcore_barrier()

    for k in range(NCH):
        blk, off = divmod(k, CPB)
        if off == CPB - 3 and blk + 1 < NBLK:
            wait_stage(blk + 1)
        wait_gather(k)
        start_scatter(k)
        if off == 4 and 1 <= blk and blk + 1 < NBLK:
            stage(blk + 1)
        if k + 3 < NCH:
            if k - 2 >= 0:
                wait_scatter(k - 2)
            start_gather(k + 3)
    for k in range(NCH - 5, NCH):
        wait_scatter(k)
    plsc.subcore_barrier()

    nseg = ROWS_PER_TILE // CHUNK + (1 if rem else 0)
    pushes = []
    for k in range(nseg):
        nr = CHUNK if k < ROWS_PER_TILE // CHUNK else rem
        r = row0 + k * CHUNK
        buf = rows[k % 2].at[pl.ds(0, nr)]
        sem = gsem[k % 2]
        if k >= 2:
            pushes[k - 2].wait()
        pltpu.sync_copy(acc.at[pl.ds(r, nr)], buf)
        pushes.append(
            pltpu.async_copy(buf, out_hbm.at[c, pl.ds(r, nr)], sem))
    for p in pushes[-2:]:
        p.wait()


_seg_sum = pl.kernel(
    _seg_sum_body,
    out_type=jax.ShapeDtypeStruct((NC, R, D), jnp.float32),
    mesh=plsc.VectorSubcoreMesh(core_axis_name="c", subcore_axis_name="s",
                                num_cores=NC, num_subcores=NS),
    scratch_types=[
        pltpu.VMEM((CPB, CHUNK), jnp.int32),
        pltpu.VMEM((CPB, CHUNK), jnp.int32),
        pltpu.VMEM((CPB, CHUNK), jnp.int32),
        pltpu.VMEM((CPB, CHUNK), jnp.int32),
        pltpu.VMEM((CHUNK, D), jnp.float32),
        pltpu.VMEM((CHUNK, D), jnp.float32),
        pltpu.VMEM((CHUNK, D), jnp.float32),
        pltpu.VMEM((CHUNK, D), jnp.float32),
        pltpu.VMEM((CHUNK, D), jnp.float32),
        pltpu.VMEM_SHARED((R, D), jnp.float32),
        pltpu.SemaphoreType.DMA,
        pltpu.SemaphoreType.DMA,
        pltpu.SemaphoreType.DMA,
        pltpu.SemaphoreType.DMA,
        pltpu.SemaphoreType.DMA,
        pltpu.SemaphoreType.DMA,
        pltpu.SemaphoreType.DMA,
        pltpu.SemaphoreType.DMA,
        pltpu.SemaphoreType.DMA,
        pltpu.SemaphoreType.DMA,
        pltpu.SemaphoreType.DMA,
        pltpu.SemaphoreType.DMA,
    ],
)


def _mlp_body(x_ref, p_ref, w_ref, b_ref, o_ref, *, relu):
    acc = x_ref[...] + p_ref[0] + p_ref[1]
    h = jnp.dot(acc, w_ref[...], preferred_element_type=jnp.float32)
    h = h + b_ref[...]
    if relu:
        h = jnp.maximum(h, 0.0)
    o_ref[...] = h


def _mlp(x, partials, w, b2d, relu):
    blk = 2000
    grid = (N_NODES // blk,)
    return pl.pallas_call(
        functools.partial(_mlp_body, relu=relu),
        grid=grid,
        in_specs=[
            pl.BlockSpec((blk, D), lambda i: (i, 0)),
            pl.BlockSpec((NC, blk, D), lambda i: (0, i, 0)),
            pl.BlockSpec((D, D), lambda i: (0, 0)),
            pl.BlockSpec((1, D), lambda i: (0, 0)),
        ],
        out_specs=pl.BlockSpec((blk, D), lambda i: (i, 0)),
        out_shape=jax.ShapeDtypeStruct((N_NODES, D), jnp.float32),
    )(x, partials, w, b2d)


def kernel(x, edge_index, W1, b1, W2, b2):
    src = edge_index[0].astype(jnp.int32)
    dst = edge_index[1].astype(jnp.int32)
    pad = E_PAD - N_EDGES
    pad_ar = jnp.arange(pad, dtype=jnp.int32)
    srcs = jnp.concatenate([src, pad_ar % N_NODES]).reshape(-1, CHUNK)
    dsts = jnp.concatenate([dst, N_NODES + pad_ar % (R - N_NODES)]).reshape(-1, CHUNK)
    b1r = b1.reshape(1, D)
    b2r = b2.reshape(1, D)

    p1 = _seg_sum(x, srcs, dsts)
    h = _mlp(x, p1, W1, b1r, relu=True)
    p2 = _seg_sum(h, srcs, dsts)
    return _mlp(h, p2, W2, b2r, relu=False)


# speedup vs baseline: 1.0610x; 1.0051x over previous
import functools

import jax
import jax.numpy as jnp
from jax import lax
from jax.experimental import pallas as pl
from jax.experimental.pallas import tpu as pltpu
from jax.experimental.pallas import tpu_sc as plsc

N_NODES = 10000
N_EDGES = 320000
D = 128

NC = 2
NS = 16
L = 16

R = 10112
CHUNK = 64
NCH = 160
CPB = 16
NBLK = NCH // CPB
NBUF = 5
E_PAD = NC * NS * NCH * CHUNK
ROWS_PER_TILE = R // NS


def _seg_sum_body(x_hbm, srcs_hbm, dsts_hbm, out_hbm,
                  src_a, dst_a, src_b, dst_b,
                  r0, r1, r2, r3, r4, acc,
                  g0, g1, g2, g3, g4, s0, s1, s2, s3, s4, ia, ib):
    c = lax.axis_index("c")
    tid = lax.axis_index("s")
    wid = c * NS + tid
    rows = (r0, r1, r2, r3, r4)
    gsem = (g0, g1, g2, g3, g4)
    ssem = (s0, s1, s2, s3, s4)
    slots = ((src_a, dst_a, ia), (src_b, dst_b, ib))

    def stage(B):
        sv, dv, sem = slots[B % 2]
        base = wid * NCH + B * CPB
        pltpu.async_copy(srcs_hbm.at[pl.ds(base, CPB)], sv, sem)
        pltpu.async_copy(dsts_hbm.at[pl.ds(base, CPB)], dv, sem)

    def wait_stage(B):
        sv, dv, sem = slots[B % 2]
        pltpu.make_async_copy(srcs_hbm.at[pl.ds(0, CPB)], sv, sem).wait()
        pltpu.make_async_copy(dsts_hbm.at[pl.ds(0, CPB)], dv, sem).wait()

    def iref(k):
        sv, dv, _ = slots[(k // CPB) % 2]
        return sv.at[k % CPB], dv.at[k % CPB]

    def start_gather(k):
        si, _ = iref(k)
        pltpu.async_copy(x_hbm.at[si], rows[k % NBUF], gsem[k % NBUF])

    def wait_gather(k):
        si, _ = iref(k)
        pltpu.make_async_copy(x_hbm.at[si], rows[k % NBUF],
                              gsem[k % NBUF]).wait()

    def start_scatter(k):
        _, di = iref(k)
        pltpu.async_copy(rows[k % NBUF], acc.at[di], ssem[k % NBUF], add=True)

    def wait_scatter(k):
        _, di = iref(k)
        pltpu.make_async_copy(rows[k % NBUF], acc.at[di],
                              ssem[k % NBUF]).wait()

    stage(0)
    stage(1)

    zero = jnp.zeros((L,), jnp.float32)

    def zrow(i, _):
        for j in range(D // L):
            r0[i, pl.ds(j * L, L)] = zero
        return 0

    lax.fori_loop(0, CHUNK, zrow, 0)
    wait_stage(0)
    start_gather(1)
    start_gather(2)
    row0 = tid * ROWS_PER_TILE
    rem = ROWS_PER_TILE % CHUNK
    for k in range(ROWS_PER_TILE // CHUNK):
        pltpu.sync_copy(r0, acc.at[pl.ds(row0 + k * CHUNK, CHUNK)])
    if rem:
        pltpu.sync_copy(r0.at[pl.ds(0, rem)],
                        acc.at[pl.ds(row0 + ROWS_PER_TILE - rem, rem)])
    start_gather(0)
    plsc.subcore_barrier()

    for k in range(NCH):
        blk, off = divmod(k, CPB)
        if off == CPB - 3 and blk + 1 < NBLK:
            wait_stage(blk + 1)
        wait_gather(k)
        start_scatter(k)
        if off == 4 and 1 <= blk and blk + 1 < NBLK:
            stage(blk + 1)
        if k + 3 < NCH:
            if k - 2 >= 0:
                wait_scatter(k - 2)
            start_gather(k + 3)
    for k in range(NCH - 5, NCH):
        wait_scatter(k)
    plsc.subcore_barrier()

    nseg = ROWS_PER_TILE // CHUNK + (1 if rem else 0)
    pushes = []
    for k in range(nseg):
        nr = CHUNK if k < ROWS_PER_TILE // CHUNK else rem
        r = row0 + k * CHUNK
        buf = rows[k % 2].at[pl.ds(0, nr)]
        sem = gsem[k % 2]
        if k >= 2:
            pushes[k - 2].wait()
        pltpu.sync_copy(acc.at[pl.ds(r, nr)], buf)
        pushes.append(
            pltpu.async_copy(buf, out_hbm.at[c, pl.ds(r, nr)], sem))
    for p in pushes[-2:]:
        p.wait()


_seg_sum = pl.kernel(
    _seg_sum_body,
    out_type=jax.ShapeDtypeStruct((NC, R, D), jnp.float32),
    mesh=plsc.VectorSubcoreMesh(core_axis_name="c", subcore_axis_name="s",
                                num_cores=NC, num_subcores=NS),
    scratch_types=[
        pltpu.VMEM((CPB, CHUNK), jnp.int32),
        pltpu.VMEM((CPB, CHUNK), jnp.int32),
        pltpu.VMEM((CPB, CHUNK), jnp.int32),
        pltpu.VMEM((CPB, CHUNK), jnp.int32),
        pltpu.VMEM((CHUNK, D), jnp.float32),
        pltpu.VMEM((CHUNK, D), jnp.float32),
        pltpu.VMEM((CHUNK, D), jnp.float32),
        pltpu.VMEM((CHUNK, D), jnp.float32),
        pltpu.VMEM((CHUNK, D), jnp.float32),
        pltpu.VMEM_SHARED((R, D), jnp.float32),
        pltpu.SemaphoreType.DMA,
        pltpu.SemaphoreType.DMA,
        pltpu.SemaphoreType.DMA,
        pltpu.SemaphoreType.DMA,
        pltpu.SemaphoreType.DMA,
        pltpu.SemaphoreType.DMA,
        pltpu.SemaphoreType.DMA,
        pltpu.SemaphoreType.DMA,
        pltpu.SemaphoreType.DMA,
        pltpu.SemaphoreType.DMA,
        pltpu.SemaphoreType.DMA,
        pltpu.SemaphoreType.DMA,
    ],
)


def _mlp_body(x_ref, p_ref, w_ref, b_ref, o_ref, *, relu):
    acc = x_ref[...] + p_ref[0] + p_ref[1]
    h = jnp.dot(acc, w_ref[...], preferred_element_type=jnp.float32)
    h = h + b_ref[...]
    if relu:
        h = jnp.maximum(h, 0.0)
    o_ref[...] = h


def _mlp(x, partials, w, b2d, relu):
    blk = 10000
    grid = (N_NODES // blk,)
    return pl.pallas_call(
        functools.partial(_mlp_body, relu=relu),
        grid=grid,
        in_specs=[
            pl.BlockSpec((blk, D), lambda i: (i, 0)),
            pl.BlockSpec((NC, blk, D), lambda i: (0, i, 0)),
            pl.BlockSpec((D, D), lambda i: (0, 0)),
            pl.BlockSpec((1, D), lambda i: (0, 0)),
        ],
        out_specs=pl.BlockSpec((blk, D), lambda i: (i, 0)),
        out_shape=jax.ShapeDtypeStruct((N_NODES, D), jnp.float32),
    )(x, partials, w, b2d)


def kernel(x, edge_index, W1, b1, W2, b2):
    src = edge_index[0].astype(jnp.int32)
    dst = edge_index[1].astype(jnp.int32)
    pad = E_PAD - N_EDGES
    pad_ar = jnp.arange(pad, dtype=jnp.int32)
    srcs = jnp.concatenate([src, pad_ar % N_NODES]).reshape(-1, CHUNK)
    dsts = jnp.concatenate([dst, N_NODES + pad_ar % (R - N_NODES)]).reshape(-1, CHUNK)
    b1r = b1.reshape(1, D)
    b2r = b2.reshape(1, D)

    p1 = _seg_sum(x, srcs, dsts)
    h = _mlp(x, p1, W1, b1r, relu=True)
    p2 = _seg_sum(h, srcs, dsts)
    return _mlp(h, p2, W2, b2r, relu=False)


# speedup vs baseline: 1.1004x; 1.0371x over previous
import functools

import jax
import jax.numpy as jnp
from jax import lax
from jax.experimental import pallas as pl
from jax.experimental.pallas import tpu as pltpu
from jax.experimental.pallas import tpu_sc as plsc

N_NODES = 10000
N_EDGES = 320000
D = 128

NC = 2
NS = 16
L = 16

R = 10112
CHUNK = 64
NCH = 160
CPB = 16
NBLK = NCH // CPB
NBUF = 5
E_PAD = NC * NS * NCH * CHUNK
ROWS_PER_TILE = R // NS


def _seg_sum_body(x_hbm, srcs_hbm, dsts_hbm, out_hbm,
                  src_a, dst_a, src_b, dst_b,
                  r0, r1, r2, r3, r4, acc,
                  g0, g1, g2, g3, g4, s0, s1, s2, s3, s4, ia, ib):
    c = lax.axis_index("c")
    tid = lax.axis_index("s")
    wid = c * NS + tid
    rows = (r0, r1, r2, r3, r4)
    gsem = (g0, g1, g2, g3, g4)
    ssem = (s0, s1, s2, s3, s4)
    slots = ((src_a, dst_a, ia), (src_b, dst_b, ib))

    def stage(B):
        sv, dv, sem = slots[B % 2]
        base = wid * NCH + B * CPB
        pltpu.async_copy(srcs_hbm.at[pl.ds(base, CPB)], sv, sem)
        pltpu.async_copy(dsts_hbm.at[pl.ds(base, CPB)], dv, sem)

    def wait_stage(B):
        sv, dv, sem = slots[B % 2]
        pltpu.make_async_copy(srcs_hbm.at[pl.ds(0, CPB)], sv, sem).wait()
        pltpu.make_async_copy(dsts_hbm.at[pl.ds(0, CPB)], dv, sem).wait()

    def iref(k):
        sv, dv, _ = slots[(k // CPB) % 2]
        return sv.at[k % CPB], dv.at[k % CPB]

    def start_gather(k):
        si, _ = iref(k)
        pltpu.async_copy(x_hbm.at[si], rows[k % NBUF], gsem[k % NBUF])

    def wait_gather(k):
        si, _ = iref(k)
        pltpu.make_async_copy(x_hbm.at[si], rows[k % NBUF],
                              gsem[k % NBUF]).wait()

    def start_scatter(k):
        _, di = iref(k)
        pltpu.async_copy(rows[k % NBUF], acc.at[di], ssem[k % NBUF], add=True)

    def wait_scatter(k):
        _, di = iref(k)
        pltpu.make_async_copy(rows[k % NBUF], acc.at[di],
                              ssem[k % NBUF]).wait()

    stage(0)
    stage(1)

    zero = jnp.zeros((L,), jnp.float32)

    def zrow(i, _):
        for j in range(D // L):
            r0[i, pl.ds(j * L, L)] = zero
        return 0

    lax.fori_loop(0, CHUNK, zrow, 0)
    wait_stage(0)
    start_gather(1)
    start_gather(2)
    row0 = tid * ROWS_PER_TILE
    rem = ROWS_PER_TILE % CHUNK
    for k in range(ROWS_PER_TILE // CHUNK):
        pltpu.sync_copy(r0, acc.at[pl.ds(row0 + k * CHUNK, CHUNK)])
    if rem:
        pltpu.sync_copy(r0.at[pl.ds(0, rem)],
                        acc.at[pl.ds(row0 + ROWS_PER_TILE - rem, rem)])
    start_gather(0)
    start_gather(3)
    plsc.subcore_barrier()

    for k in range(NCH):
        blk, off = divmod(k, CPB)
        if off == CPB - 3 and blk + 1 < NBLK:
            wait_stage(blk + 1)
        wait_gather(k)
        start_scatter(k)
        if off == 4 and 1 <= blk and blk + 1 < NBLK:
            stage(blk + 1)
        if k + 4 < NCH:
            if k - 1 >= 0:
                wait_scatter(k - 1)
            start_gather(k + 4)
    for k in range(NCH - 5, NCH):
        wait_scatter(k)
    plsc.subcore_barrier()

    nseg = ROWS_PER_TILE // CHUNK + (1 if rem else 0)
    pushes = []
    for k in range(nseg):
        nr = CHUNK if k < ROWS_PER_TILE // CHUNK else rem
        r = row0 + k * CHUNK
        buf = rows[k % 2].at[pl.ds(0, nr)]
        sem = gsem[k % 2]
        if k >= 2:
            pushes[k - 2].wait()
        pltpu.sync_copy(acc.at[pl.ds(r, nr)], buf)
        pushes.append(
            pltpu.async_copy(buf, out_hbm.at[c, pl.ds(r, nr)], sem))
    for p in pushes[-2:]:
        p.wait()


_seg_sum = pl.kernel(
    _seg_sum_body,
    out_type=jax.ShapeDtypeStruct((NC, R, D), jnp.float32),
    mesh=plsc.VectorSubcoreMesh(core_axis_name="c", subcore_axis_name="s",
                                num_cores=NC, num_subcores=NS),
    scratch_types=[
        pltpu.VMEM((CPB, CHUNK), jnp.int32),
        pltpu.VMEM((CPB, CHUNK), jnp.int32),
        pltpu.VMEM((CPB, CHUNK), jnp.int32),
        pltpu.VMEM((CPB, CHUNK), jnp.int32),
        pltpu.VMEM((CHUNK, D), jnp.float32),
        pltpu.VMEM((CHUNK, D), jnp.float32),
        pltpu.VMEM((CHUNK, D), jnp.float32),
        pltpu.VMEM((CHUNK, D), jnp.float32),
        pltpu.VMEM((CHUNK, D), jnp.float32),
        pltpu.VMEM_SHARED((R, D), jnp.float32),
        pltpu.SemaphoreType.DMA,
        pltpu.SemaphoreType.DMA,
        pltpu.SemaphoreType.DMA,
        pltpu.SemaphoreType.DMA,
        pltpu.SemaphoreType.DMA,
        pltpu.SemaphoreType.DMA,
        pltpu.SemaphoreType.DMA,
        pltpu.SemaphoreType.DMA,
        pltpu.SemaphoreType.DMA,
        pltpu.SemaphoreType.DMA,
        pltpu.SemaphoreType.DMA,
        pltpu.SemaphoreType.DMA,
    ],
)


def _mlp_body(x_ref, p_ref, w_ref, b_ref, o_ref, *, relu):
    acc = x_ref[...] + p_ref[0] + p_ref[1]
    h = jnp.dot(acc, w_ref[...], preferred_element_type=jnp.float32)
    h = h + b_ref[...]
    if relu:
        h = jnp.maximum(h, 0.0)
    o_ref[...] = h


def _mlp(x, partials, w, b2d, relu):
    blk = 10000
    grid = (N_NODES // blk,)
    return pl.pallas_call(
        functools.partial(_mlp_body, relu=relu),
        grid=grid,
        in_specs=[
            pl.BlockSpec((blk, D), lambda i: (i, 0)),
            pl.BlockSpec((NC, blk, D), lambda i: (0, i, 0)),
            pl.BlockSpec((D, D), lambda i: (0, 0)),
            pl.BlockSpec((1, D), lambda i: (0, 0)),
        ],
        out_specs=pl.BlockSpec((blk, D), lambda i: (i, 0)),
        out_shape=jax.ShapeDtypeStruct((N_NODES, D), jnp.float32),
    )(x, partials, w, b2d)


def kernel(x, edge_index, W1, b1, W2, b2):
    src = edge_index[0].astype(jnp.int32)
    dst = edge_index[1].astype(jnp.int32)
    pad = E_PAD - N_EDGES
    pad_ar = jnp.arange(pad, dtype=jnp.int32)
    srcs = jnp.concatenate([src, pad_ar % N_NODES]).reshape(-1, CHUNK)
    dsts = jnp.concatenate([dst, N_NODES + pad_ar % (R - N_NODES)]).reshape(-1, CHUNK)
    b1r = b1.reshape(1, D)
    b2r = b2.reshape(1, D)

    p1 = _seg_sum(x, srcs, dsts)
    h = _mlp(x, p1, W1, b1r, relu=True)
    p2 = _seg_sum(h, srcs, dsts)
    return _mlp(h, p2, W2, b2r, relu=False)


# speedup vs baseline: 1.1045x; 1.0038x over previous
import functools

import jax
import jax.numpy as jnp
from jax import lax
from jax.experimental import pallas as pl
from jax.experimental.pallas import tpu as pltpu
from jax.experimental.pallas import tpu_sc as plsc

N_NODES = 10000
N_EDGES = 320000
D = 128

NC = 2
NS = 16
L = 16

R = 10112
CHUNK = 64
NCH = 160
CPB = 16
NBLK = NCH // CPB
NBUF = 5
E_PAD = NC * NS * NCH * CHUNK
ROWS_PER_TILE = R // NS


def _seg_sum_body(x_hbm, srcs_hbm, dsts_hbm, out_hbm,
                  src_a, dst_a, src_b, dst_b,
                  r0, r1, r2, r3, r4, acc,
                  g0, g1, g2, g3, g4, s0, s1, s2, s3, s4, ia, ib):
    c = lax.axis_index("c")
    tid = lax.axis_index("s")
    wid = c * NS + tid
    rows = (r0, r1, r2, r3, r4)
    gsem = (g0, g1, g2, g3, g4)
    ssem = (s0, s1, s2, s3, s4)
    slots = ((src_a, dst_a, ia), (src_b, dst_b, ib))

    def stage(B):
        sv, dv, sem = slots[B % 2]
        base = wid * NCH + B * CPB
        pltpu.async_copy(srcs_hbm.at[pl.ds(base, CPB)], sv, sem)
        pltpu.async_copy(dsts_hbm.at[pl.ds(base, CPB)], dv, sem)

    def wait_stage(B):
        sv, dv, sem = slots[B % 2]
        pltpu.make_async_copy(srcs_hbm.at[pl.ds(0, CPB)], sv, sem).wait()
        pltpu.make_async_copy(dsts_hbm.at[pl.ds(0, CPB)], dv, sem).wait()

    def iref(k):
        sv, dv, _ = slots[(k // CPB) % 2]
        return sv.at[k % CPB], dv.at[k % CPB]

    def start_gather(k):
        si, _ = iref(k)
        pltpu.async_copy(x_hbm.at[si], rows[k % NBUF], gsem[k % NBUF])

    def wait_gather(k):
        si, _ = iref(k)
        pltpu.make_async_copy(x_hbm.at[si], rows[k % NBUF],
                              gsem[k % NBUF]).wait()

    def start_scatter(k):
        _, di = iref(k)
        pltpu.async_copy(rows[k % NBUF], acc.at[di], ssem[k % NBUF], add=True)

    def wait_scatter(k):
        _, di = iref(k)
        pltpu.make_async_copy(rows[k % NBUF], acc.at[di],
                              ssem[k % NBUF]).wait()

    stage(0)
    stage(1)

    zero = jnp.zeros((L,), jnp.float32)

    def zrow(i, _):
        for j in range(D // L):
            r0[i, pl.ds(j * L, L)] = zero
        return 0

    lax.fori_loop(0, CHUNK, zrow, 0)
    wait_stage(0)
    start_gather(1)
    start_gather(2)
    row0 = tid * ROWS_PER_TILE
    rem = ROWS_PER_TILE % CHUNK
    for k in range(ROWS_PER_TILE // CHUNK):
        pltpu.sync_copy(r0, acc.at[pl.ds(row0 + k * CHUNK, CHUNK)])
    if rem:
        pltpu.sync_copy(r0.at[pl.ds(0, rem)],
                        acc.at[pl.ds(row0 + ROWS_PER_TILE - rem, rem)])
    start_gather(0)
    start_gather(3)
    plsc.subcore_barrier()

    for k in range(NCH):
        blk, off = divmod(k, CPB)
        if off == CPB - 4 and blk + 1 < NBLK:
            wait_stage(blk + 1)
        wait_gather(k)
        start_scatter(k)
        if off == 4 and 1 <= blk and blk + 1 < NBLK:
            stage(blk + 1)
        if k + 4 < NCH:
            if k - 1 >= 0:
                wait_scatter(k - 1)
            start_gather(k + 4)
    for k in range(NCH - 5, NCH):
        wait_scatter(k)
    plsc.subcore_barrier()

    nseg = ROWS_PER_TILE // CHUNK + (1 if rem else 0)
    pushes = []
    for k in range(nseg):
        nr = CHUNK if k < ROWS_PER_TILE // CHUNK else rem
        r = row0 + k * CHUNK
        buf = rows[k % 2].at[pl.ds(0, nr)]
        sem = gsem[k % 2]
        if k >= 2:
            pushes[k - 2].wait()
        pltpu.sync_copy(acc.at[pl.ds(r, nr)], buf)
        pushes.append(
            pltpu.async_copy(buf, out_hbm.at[c, pl.ds(r, nr)], sem))
    for p in pushes[-2:]:
        p.wait()


_seg_sum = pl.kernel(
    _seg_sum_body,
    out_type=jax.ShapeDtypeStruct((NC, R, D), jnp.float32),
    mesh=plsc.VectorSubcoreMesh(core_axis_name="c", subcore_axis_name="s",
                                num_cores=NC, num_subcores=NS),
    scratch_types=[
        pltpu.VMEM((CPB, CHUNK), jnp.int32),
        pltpu.VMEM((CPB, CHUNK), jnp.int32),
        pltpu.VMEM((CPB, CHUNK), jnp.int32),
        pltpu.VMEM((CPB, CHUNK), jnp.int32),
        pltpu.VMEM((CHUNK, D), jnp.float32),
        pltpu.VMEM((CHUNK, D), jnp.float32),
        pltpu.VMEM((CHUNK, D), jnp.float32),
        pltpu.VMEM((CHUNK, D), jnp.float32),
        pltpu.VMEM((CHUNK, D), jnp.float32),
        pltpu.VMEM_SHARED((R, D), jnp.float32),
        pltpu.SemaphoreType.DMA,
        pltpu.SemaphoreType.DMA,
        pltpu.SemaphoreType.DMA,
        pltpu.SemaphoreType.DMA,
        pltpu.SemaphoreType.DMA,
        pltpu.SemaphoreType.DMA,
        pltpu.SemaphoreType.DMA,
        pltpu.SemaphoreType.DMA,
        pltpu.SemaphoreType.DMA,
        pltpu.SemaphoreType.DMA,
        pltpu.SemaphoreType.DMA,
        pltpu.SemaphoreType.DMA,
    ],
)


def _mlp_body(x_ref, p_ref, w_ref, b_ref, o_ref, *, relu):
    acc = x_ref[...] + p_ref[0] + p_ref[1]
    h = jnp.dot(acc, w_ref[...], preferred_element_type=jnp.float32)
    h = h + b_ref[...]
    if relu:
        h = jnp.maximum(h, 0.0)
    o_ref[...] = h


def _mlp(x, partials, w, b2d, relu):
    blk = 10000
    grid = (N_NODES // blk,)
    return pl.pallas_call(
        functools.partial(_mlp_body, relu=relu),
        grid=grid,
        in_specs=[
            pl.BlockSpec((blk, D), lambda i: (i, 0)),
            pl.BlockSpec((NC, blk, D), lambda i: (0, i, 0)),
            pl.BlockSpec((D, D), lambda i: (0, 0)),
            pl.BlockSpec((1, D), lambda i: (0, 0)),
        ],
        out_specs=pl.BlockSpec((blk, D), lambda i: (i, 0)),
        out_shape=jax.ShapeDtypeStruct((N_NODES, D), jnp.float32),
    )(x, partials, w, b2d)


def kernel(x, edge_index, W1, b1, W2, b2):
    src = edge_index[0].astype(jnp.int32)
    dst = edge_index[1].astype(jnp.int32)
    pad = E_PAD - N_EDGES
    pad_ar = jnp.arange(pad, dtype=jnp.int32)
    srcs = jnp.concatenate([src, pad_ar % N_NODES]).reshape(-1, CHUNK)
    dsts = jnp.concatenate([dst, N_NODES + pad_ar % (R - N_NODES)]).reshape(-1, CHUNK)
    b1r = b1.reshape(1, D)
    b2r = b2.reshape(1, D)

    p1 = _seg_sum(x, srcs, dsts)
    h = _mlp(x, p1, W1, b1r, relu=True)
    p2 = _seg_sum(h, srcs, dsts)
    return _mlp(h, p2, W2, b2r, relu=False)
